# baseline (speedup 1.0000x reference)
import functools

import jax
import jax.numpy as jnp
from jax import lax
from jax.experimental import pallas as pl
from jax.experimental.pallas import tpu as pltpu
from jax.experimental.pallas import tpu_sc as plsc

E = 16
K = 2
B = 4096
D = 1024

BT = 512
NC, NS, L = 2, 16, 16
NW = NC * NS
TPW = B // NW
CH = 16
NCH = TPW // CH
NROW = BT // 128


def _gate_kernel(x_ref, w_ref, b_ref, r0_ref, r1_ref, wa_ref, wb_ref,
                 st_ref, soft_ref, hard_ref):
    i = pl.program_id(0)
    lg = lax.dot_general(w_ref[...], x_ref[...], (((1,), (1,)), ((), ())),
                         preferred_element_type=jnp.float32)
    lg = lg + b_ref[...]
    iota = lax.broadcasted_iota(jnp.int32, (E, BT), 0)
    m1 = jnp.max(lg, axis=0, keepdims=True)
    i1 = jnp.min(jnp.where(lg == m1, iota, E), axis=0, keepdims=True)
    masked = jnp.where(iota == i1, -jnp.inf, lg)
    m2 = jnp.max(masked, axis=0, keepdims=True)
    i2 = jnp.min(jnp.where(masked == m2, iota, E), axis=0, keepdims=True)
    e21 = jnp.exp(m2 - m1)
    den = 1.0 / (1.0 + e21)
    wa = den
    wb = e21 * den
    g = jnp.where(iota == i1, wa, 0.0) + jnp.where(iota == i2, wb, 0.0)
    s = jnp.where(g < 1e-5, 1.0, 0.0)
    bvec = i * BT + lax.broadcasted_iota(jnp.int32, (1, BT), 1)
    r0 = i1 * B + bvec
    r1 = i2 * B + bvec
    for j in range(NROW):
        cols = slice(128 * j, 128 * (j + 1))
        row = pl.ds(i * NROW + j, 1)
        r0_ref[row, :] = r0[:, cols]
        r1_ref[row, :] = r1[:, cols]
        wa_ref[row, :] = wa[:, cols]
        wb_ref[row, :] = wb[:, cols]
    st_ref[...] = s

    @pl.when(i == 0)
    def _():
        soft_ref[...] = jnp.zeros_like(soft_ref)
        hard_ref[...] = jnp.zeros_like(hard_ref)

    soft_ref[...] += jnp.sum(g, axis=1, keepdims=True) * (1.0 / B)
    hard_ref[...] += jnp.sum(1.0 - s, axis=1, keepdims=True) * (1.0 / B)


def _gate(x, W, b):
    return pl.pallas_call(
        _gate_kernel,
        grid=(B // BT,),
        in_specs=[
            pl.BlockSpec((BT, D), lambda i: (i, 0)),
            pl.BlockSpec((E, D), lambda i: (0, 0)),
            pl.BlockSpec((E, 1), lambda i: (0, 0)),
        ],
        out_specs=[
            pl.BlockSpec((NW, 128), lambda i: (0, 0)),
            pl.BlockSpec((NW, 128), lambda i: (0, 0)),
            pl.BlockSpec((NW, 128), lambda i: (0, 0)),
            pl.BlockSpec((NW, 128), lambda i: (0, 0)),
            pl.BlockSpec((E, BT), lambda i: (0, i)),
            pl.BlockSpec((E, 1), lambda i: (0, 0)),
            pl.BlockSpec((E, 1), lambda i: (0, 0)),
        ],
        out_shape=[
            jax.ShapeDtypeStruct((NW, 128), jnp.int32),
            jax.ShapeDtypeStruct((NW, 128), jnp.int32),
            jax.ShapeDtypeStruct((NW, 128), jnp.float32),
            jax.ShapeDtypeStruct((NW, 128), jnp.float32),
            jax.ShapeDtypeStruct((E, B), jnp.float32),
            jax.ShapeDtypeStruct((E, 1), jnp.float32),
            jax.ShapeDtypeStruct((E, 1), jnp.float32),
        ],
        compiler_params=pltpu.CompilerParams(
            dimension_semantics=("arbitrary",)),
    )(x, W, b.reshape(E, 1))


NBUF = 3


def _combine_body(f_hbm, r0_hbm, r1_hbm, wa_hbm, wb_hbm, y_hbm,
                  idx0_v, idx1_v, wa_v, wb_v,
                  ra0, ra1, rb0, rb1, rc0, rc1,
                  sga, sgb, sgc, sya, syb, syc):
    wid = lax.axis_index("s") * NC + lax.axis_index("c")
    base = wid * TPW
    pltpu.sync_copy(r0_hbm.at[wid], idx0_v)
    pltpu.sync_copy(r1_hbm.at[wid], idx1_v)
    pltpu.sync_copy(wa_hbm.at[wid], wa_v)
    pltpu.sync_copy(wb_hbm.at[wid], wb_v)

    bufs = [(ra0, ra1, sga, sya), (rb0, rb1, sgb, syb), (rc0, rc1, sgc, syc)]

    def start_gather(c, p):
        r0b, r1b, sg, _ = bufs[p]
        cp0 = pltpu.async_copy(
            f_hbm.at[idx0_v.at[pl.ds(c * CH, CH)]], r0b, sg)
        cp1 = pltpu.async_copy(
            f_hbm.at[idx1_v.at[pl.ds(c * CH, CH)]], r1b, sg)
        return cp0, cp1

    pending = {0: start_gather(0, 0), 1: start_gather(1, 1)}
    ywrites = [None] * NBUF
    for c in range(NCH):
        p = c % NBUF
        r0b, r1b, _, sy = bufs[p]
        cp0, cp1 = pending.pop(c)
        cp0.wait()
        cp1.wait()
        if c + 2 < NCH:
            pn = (c + 2) % NBUF
            yw = ywrites[pn]
            if yw is not None:
                yw.wait()
                ywrites[pn] = None
            pending[c + 2] = start_gather(c + 2, pn)
        wvec0 = wa_v[pl.ds(c * L, L)]
        wvec1 = wb_v[pl.ds(c * L, L)]
        gdn = lax.GatherDimensionNumbers(
            offset_dims=(), collapsed_slice_dims=(0,), start_index_map=(0,))
        for t in range(CH):
            sel = jnp.full((L, 1), t, jnp.int32)
            sw0 = lax.gather(wvec0, sel, gdn, (1,),
                             mode=lax.GatherScatterMode.PROMISE_IN_BOUNDS)
            sw1 = lax.gather(wvec1, sel, gdn, (1,),
                             mode=lax.GatherScatterMode.PROMISE_IN_BOUNDS)

            def body(j, _, t=t, sw0=sw0, sw1=sw1, r0b=r0b, r1b=r1b):
                a = r0b[t, pl.ds(j * L, L)]
                bvals = r1b[t, pl.ds(j * L, L)]
                r0b[t, pl.ds(j * L, L)] = sw0 * a + sw1 * bvals
                return 0

            lax.fori_loop(0, D // L, body, 0, unroll=4)
        yw = pltpu.make_async_copy(
            r0b, y_hbm.at[pl.ds(base + c * CH, CH)], sy)
        yw.start()
        ywrites[p] = yw
    for yw in ywrites:
        if yw is not None:
            yw.wait()


@functools.cache
def _make_combine():
    return functools.partial(
        pl.kernel,
        out_type=jax.ShapeDtypeStruct((B, D), jnp.float32),
        mesh=plsc.VectorSubcoreMesh(core_axis_name="c", subcore_axis_name="s",
                                    num_cores=NC, num_subcores=NS),
        scratch_types=[
            pltpu.VMEM((TPW,), jnp.int32),
            pltpu.VMEM((TPW,), jnp.int32),
            pltpu.VMEM((TPW,), jnp.float32),
            pltpu.VMEM((TPW,), jnp.float32),
            pltpu.VMEM((CH, D), jnp.float32),
            pltpu.VMEM((CH, D), jnp.float32),
            pltpu.VMEM((CH, D), jnp.float32),
            pltpu.VMEM((CH, D), jnp.float32),
            pltpu.VMEM((CH, D), jnp.float32),
            pltpu.VMEM((CH, D), jnp.float32),
            pltpu.SemaphoreType.DMA,
            pltpu.SemaphoreType.DMA,
            pltpu.SemaphoreType.DMA,
            pltpu.SemaphoreType.DMA,
            pltpu.SemaphoreType.DMA,
            pltpu.SemaphoreType.DMA,
        ],
    )(_combine_body)


@jax.jit
def kernel(f, x, W, b):
    r0, r1, wa, wb, st, soft, hard = _gate(x, W, b)
    y = _make_combine()(f.reshape(E * B, D), r0, r1, wa, wb)
    return (y, soft, hard, st.T.reshape(B, E, 1))


# speedup vs baseline: 1.0364x; 1.0364x over previous
import functools

import jax
import jax.numpy as jnp
from jax import lax
from jax.experimental import pallas as pl
from jax.experimental.pallas import tpu as pltpu
from jax.experimental.pallas import tpu_sc as plsc

E = 16
K = 2
B = 4096
D = 1024

BT = 1024
NC, NS, L = 2, 16, 16
NW = NC * NS
TPW = B // NW
CH = 16
NCH = TPW // CH
NROW = BT // 128


def _gate_kernel(x_ref, w_ref, b_ref, r0_ref, r1_ref, wa_ref, wb_ref,
                 st_ref, soft_ref, hard_ref):
    i = pl.program_id(0)
    lg = lax.dot_general(w_ref[...], x_ref[...], (((1,), (1,)), ((), ())),
                         preferred_element_type=jnp.float32)
    lg = lg + b_ref[...]
    iota = lax.broadcasted_iota(jnp.int32, (E, BT), 0)
    m1 = jnp.max(lg, axis=0, keepdims=True)
    i1 = jnp.min(jnp.where(lg == m1, iota, E), axis=0, keepdims=True)
    masked = jnp.where(iota == i1, -jnp.inf, lg)
    m2 = jnp.max(masked, axis=0, keepdims=True)
    i2 = jnp.min(jnp.where(masked == m2, iota, E), axis=0, keepdims=True)
    e21 = jnp.exp(m2 - m1)
    den = 1.0 / (1.0 + e21)
    wa = den
    wb = e21 * den
    g = jnp.where(iota == i1, wa, 0.0) + jnp.where(iota == i2, wb, 0.0)
    s = jnp.where(g < 1e-5, 1.0, 0.0)
    bvec = i * BT + lax.broadcasted_iota(jnp.int32, (1, BT), 1)
    r0 = i1 * B + bvec
    r1 = i2 * B + bvec
    for j in range(NROW):
        cols = slice(128 * j, 128 * (j + 1))
        row = pl.ds(i * NROW + j, 1)
        r0_ref[row, :] = r0[:, cols]
        r1_ref[row, :] = r1[:, cols]
        wa_ref[row, :] = wa[:, cols]
        wb_ref[row, :] = wb[:, cols]
    st_ref[...] = s

    @pl.when(i == 0)
    def _():
        soft_ref[...] = jnp.zeros_like(soft_ref)
        hard_ref[...] = jnp.zeros_like(hard_ref)

    soft_ref[...] += jnp.sum(g, axis=1, keepdims=True) * (1.0 / B)
    hard_ref[...] += jnp.sum(1.0 - s, axis=1, keepdims=True) * (1.0 / B)


def _gate(x, W, b):
    return pl.pallas_call(
        _gate_kernel,
        grid=(B // BT,),
        in_specs=[
            pl.BlockSpec((BT, D), lambda i: (i, 0)),
            pl.BlockSpec((E, D), lambda i: (0, 0)),
            pl.BlockSpec((E, 1), lambda i: (0, 0)),
        ],
        out_specs=[
            pl.BlockSpec((NW, 128), lambda i: (0, 0)),
            pl.BlockSpec((NW, 128), lambda i: (0, 0)),
            pl.BlockSpec((NW, 128), lambda i: (0, 0)),
            pl.BlockSpec((NW, 128), lambda i: (0, 0)),
            pl.BlockSpec((E, BT), lambda i: (0, i)),
            pl.BlockSpec((E, 1), lambda i: (0, 0)),
            pl.BlockSpec((E, 1), lambda i: (0, 0)),
        ],
        out_shape=[
            jax.ShapeDtypeStruct((NW, 128), jnp.int32),
            jax.ShapeDtypeStruct((NW, 128), jnp.int32),
            jax.ShapeDtypeStruct((NW, 128), jnp.float32),
            jax.ShapeDtypeStruct((NW, 128), jnp.float32),
            jax.ShapeDtypeStruct((E, B), jnp.float32),
            jax.ShapeDtypeStruct((E, 1), jnp.float32),
            jax.ShapeDtypeStruct((E, 1), jnp.float32),
        ],
        compiler_params=pltpu.CompilerParams(
            dimension_semantics=("arbitrary",)),
    )(x, W, b.reshape(E, 1))


NBUF = 2


def _combine_body(f_hbm, r0_hbm, r1_hbm, wa_hbm, wb_hbm, y_hbm,
                  idx0_v, idx1_v, wa_v, wb_v,
                  ra0, ra1, rb0, rb1,
                  sga, sgb, sya, syb):
    wid = lax.axis_index("s") * NC + lax.axis_index("c")
    base = wid * TPW
    pltpu.sync_copy(r0_hbm.at[wid], idx0_v)
    pltpu.sync_copy(r1_hbm.at[wid], idx1_v)
    pltpu.sync_copy(wa_hbm.at[wid], wa_v)
    pltpu.sync_copy(wb_hbm.at[wid], wb_v)

    bufs = [(ra0, ra1, sga, sya), (rb0, rb1, sgb, syb)]

    def start_gather(c, p):
        r0b, r1b, sg, _ = bufs[p]
        cp0 = pltpu.async_copy(
            f_hbm.at[idx0_v.at[pl.ds(c * CH, CH)]], r0b, sg)
        cp1 = pltpu.async_copy(
            f_hbm.at[idx1_v.at[pl.ds(c * CH, CH)]], r1b, sg)
        return cp0, cp1

    pending = {0: start_gather(0, 0)}
    ywrites = [None] * NBUF
    for c in range(NCH):
        p = c % NBUF
        r0b, r1b, _, sy = bufs[p]
        cp0, cp1 = pending.pop(c)
        cp0.wait()
        cp1.wait()
        if c + 1 < NCH:
            pn = (c + 1) % NBUF
            yw = ywrites[pn]
            if yw is not None:
                yw.wait()
                ywrites[pn] = None
            pending[c + 1] = start_gather(c + 1, pn)
        wvec0 = wa_v[pl.ds(c * L, L)]
        wvec1 = wb_v[pl.ds(c * L, L)]
        gdn = lax.GatherDimensionNumbers(
            offset_dims=(), collapsed_slice_dims=(0,), start_index_map=(0,))
        for t in range(CH):
            sel = jnp.full((L, 1), t, jnp.int32)
            sw0 = lax.gather(wvec0, sel, gdn, (1,),
                             mode=lax.GatherScatterMode.PROMISE_IN_BOUNDS)
            sw1 = lax.gather(wvec1, sel, gdn, (1,),
                             mode=lax.GatherScatterMode.PROMISE_IN_BOUNDS)

            def body(j, _, t=t, sw0=sw0, sw1=sw1, r0b=r0b, r1b=r1b):
                a = r0b[t, pl.ds(j * L, L)]
                bvals = r1b[t, pl.ds(j * L, L)]
                r0b[t, pl.ds(j * L, L)] = sw0 * a + sw1 * bvals
                return 0

            lax.fori_loop(0, D // L, body, 0, unroll=4)
        yw = pltpu.make_async_copy(
            r0b, y_hbm.at[pl.ds(base + c * CH, CH)], sy)
        yw.start()
        ywrites[p] = yw
    for yw in ywrites:
        if yw is not None:
            yw.wait()


@functools.cache
def _make_combine():
    return functools.partial(
        pl.kernel,
        out_type=jax.ShapeDtypeStruct((B, D), jnp.float32),
        mesh=plsc.VectorSubcoreMesh(core_axis_name="c", subcore_axis_name="s",
                                    num_cores=NC, num_subcores=NS),
        scratch_types=[
            pltpu.VMEM((TPW,), jnp.int32),
            pltpu.VMEM((TPW,), jnp.int32),
            pltpu.VMEM((TPW,), jnp.float32),
            pltpu.VMEM((TPW,), jnp.float32),
            pltpu.VMEM((CH, D), jnp.float32),
            pltpu.VMEM((CH, D), jnp.float32),
            pltpu.VMEM((CH, D), jnp.float32),
            pltpu.VMEM((CH, D), jnp.float32),
            pltpu.SemaphoreType.DMA,
            pltpu.SemaphoreType.DMA,
            pltpu.SemaphoreType.DMA,
            pltpu.SemaphoreType.DMA,
        ],
    )(_combine_body)


@jax.jit
def kernel(f, x, W, b):
    r0, r1, wa, wb, st, soft, hard = _gate(x, W, b)
    y = _make_combine()(f.reshape(E * B, D), r0, r1, wa, wb)
    return (y, soft, hard, st.T.reshape(B, E, 1))
